# baseline (speedup 1.0000x reference)
import functools

import jax
import jax.numpy as jnp
from jax import lax
from jax.experimental import pallas as pl
from jax.experimental.pallas import tpu as pltpu
from jax.experimental.pallas import tpu_sc as plsc

NC = 2
NS = 16
NW = NC * NS

E = 16
H = 1024
I = 1024
M = 256


def _route(topk_indices, topk_weights, tiles):
    T_, K_ = topk_indices.shape
    N = T_ * K_
    pad_n = tiles * M
    e_flat = topk_indices.reshape(N).astype(jnp.int32)
    order = jnp.argsort(e_flat, stable=True).astype(jnp.int32)
    sorted_e = e_flat[order]
    g = jnp.zeros((E,), jnp.int32).at[e_flat].add(1)
    seg_start = (jnp.cumsum(g) - g).astype(jnp.int32)
    padded = ((g + (M - 1)) // M) * M
    ends = jnp.cumsum(padded).astype(jnp.int32)
    base = ends - padded
    s = jnp.arange(N, dtype=jnp.int32)
    rank = s - seg_start[sorted_e]
    dest = base[sorted_e] + rank
    tok = order // K_
    token_for_slot = jnp.zeros((pad_n,), jnp.int32).at[dest].set(tok)
    ws_slot = jnp.zeros((pad_n,), jnp.float32).at[dest].set(
        topk_weights.reshape(N)[order])
    invpos = jnp.zeros((N,), jnp.int32).at[order].set(dest)
    tile_starts = jnp.arange(tiles, dtype=jnp.int32) * M
    te = jnp.searchsorted(ends, tile_starts, side="right").astype(jnp.int32)
    last_e = jnp.searchsorted(ends, ends[-1] - 1, side="right").astype(jnp.int32)
    te = jnp.where(tile_starts < ends[-1], jnp.clip(te, 0, E - 1), last_e)
    nt = (ends[-1:] // M).astype(jnp.int32)
    return token_for_slot, ws_slot, invpos, te, nt


def _sc_gather(x, idx):
    pad_n = idx.shape[0]
    per_w = pad_n // NW
    W = 32
    SUB = 16
    NB = 3
    nch = per_w // W
    mesh = plsc.VectorSubcoreMesh(core_axis_name="core", subcore_axis_name="subcore")

    @functools.partial(
        pl.kernel,
        out_type=jax.ShapeDtypeStruct((pad_n, H), x.dtype),
        mesh=mesh,
        scratch_types=[pltpu.VMEM((2 * nch, SUB), jnp.int32),
                       pltpu.VMEM((W, H), jnp.float32),
                       pltpu.VMEM((W, H), jnp.float32),
                       pltpu.VMEM((W, H), jnp.float32),
                       pltpu.SemaphoreType.DMA,
                       pltpu.SemaphoreType.DMA,
                       pltpu.SemaphoreType.DMA],
    )
    def k(x_hbm, i_hbm, o_hbm, idx_v, r0, r1, r2, s0, s1, s2):
        wid = lax.axis_index("subcore") * NC + lax.axis_index("core")
        base = wid * per_w
        pltpu.sync_copy(i_hbm.at[wid], idx_v)
        bufs = [(r0, s0), (r1, s1), (r2, s2)]

        def fire(c):
            r, s = bufs[c % NB]
            pltpu.async_copy(x_hbm.at[idx_v.at[2 * c]], r.at[pl.ds(0, SUB)], s)
            pltpu.async_copy(x_hbm.at[idx_v.at[2 * c + 1]],
                             r.at[pl.ds(SUB, SUB)], s)

        for c in range(min(NB, nch)):
            fire(c)
        for c in range(nch):
            r, s = bufs[c % NB]
            pltpu.make_async_copy(x_hbm.at[idx_v.at[2 * c]],
                                  r.at[pl.ds(0, SUB)], s).wait()
            pltpu.make_async_copy(x_hbm.at[idx_v.at[2 * c + 1]],
                                  r.at[pl.ds(SUB, SUB)], s).wait()
            pltpu.sync_copy(r, o_hbm.at[pl.ds(base + c * W, W)])
            if c + NB < nch:
                fire(c + NB)

    return k(x, idx.reshape(NW, 2 * nch, SUB))


def _sc_combine(ys, p0, p1):
    T_ = p0.shape[0]
    per_w = T_ // NW
    C = 16
    nch = per_w // C
    mesh = plsc.VectorSubcoreMesh(core_axis_name="core", subcore_axis_name="subcore")

    @functools.partial(
        pl.kernel,
        out_type=jax.ShapeDtypeStruct((T_, H), ys.dtype),
        mesh=mesh,
        scratch_types=[pltpu.VMEM((nch, C), jnp.int32),
                       pltpu.VMEM((nch, C), jnp.int32),
                       pltpu.VMEM((C, H), jnp.float32),
                       pltpu.VMEM((C, H), jnp.float32),
                       pltpu.VMEM((C, H), jnp.float32),
                       pltpu.SemaphoreType.DMA,
                       pltpu.SemaphoreType.DMA],
    )
    def k(ys_hbm, p0_hbm, p1_hbm, o_hbm, p0_v, p1_v, a_v, b_v, o_v, s0, s1):
        wid = lax.axis_index("subcore") * NC + lax.axis_index("core")
        base = wid * per_w
        pltpu.sync_copy(p0_hbm.at[wid], p0_v)
        pltpu.sync_copy(p1_hbm.at[wid], p1_v)
        for c in range(nch):
            pltpu.async_copy(ys_hbm.at[p0_v.at[c]], a_v, s0)
            pltpu.async_copy(ys_hbm.at[p1_v.at[c]], b_v, s1)
            pltpu.make_async_copy(ys_hbm.at[p0_v.at[c]], a_v, s0).wait()
            pltpu.make_async_copy(ys_hbm.at[p1_v.at[c]], b_v, s1).wait()

            @pl.loop(0, C)
            def _(r):
                @pl.loop(0, H, step=16)
                def _(col):
                    o_v[r, pl.ds(col, 16)] = (a_v[r, pl.ds(col, 16)]
                                              + b_v[r, pl.ds(col, 16)])

            pltpu.sync_copy(o_v, o_hbm.at[pl.ds(base + c * C, C)])

    return k(ys, p0.reshape(NW, nch, C), p1.reshape(NW, nch, C))


def _tc_gmm_body(te_ref, nt_ref, xs_ref, g_ref, u_ref, d_ref, ws_ref, ys_ref):
    i = pl.program_id(0)

    @pl.when(i < nt_ref[0])
    def _():
        x = xs_ref[...]
        gate = jnp.dot(x, g_ref[...], preferred_element_type=jnp.float32)
        up = jnp.dot(x, u_ref[...], preferred_element_type=jnp.float32)
        h = (gate * jax.nn.sigmoid(gate)) * up
        y = jnp.dot(h, d_ref[...], preferred_element_type=jnp.float32)
        w = ws_ref[0, 0, :]
        ys_ref[...] = y * w[:, None]


def _tc_gmm(xs, gate_up_proj, down_proj, ws3, te, nt, tiles, interpret=False):
    grid_spec = pltpu.PrefetchScalarGridSpec(
        num_scalar_prefetch=2,
        grid=(tiles,),
        in_specs=[
            pl.BlockSpec((M, H), lambda i, te, nt: (i, 0)),
            pl.BlockSpec((H, I), lambda i, te, nt: (te[i], 0)),
            pl.BlockSpec((H, I), lambda i, te, nt: (te[i], 1)),
            pl.BlockSpec((I, H), lambda i, te, nt: (te[i], 0)),
            pl.BlockSpec((1, 1, M), lambda i, te, nt: (i, 0, 0)),
        ],
        out_specs=pl.BlockSpec((M, H), lambda i, te, nt: (i, 0)),
    )
    return pl.pallas_call(
        _tc_gmm_body,
        grid_spec=grid_spec,
        out_shape=jax.ShapeDtypeStruct((tiles * M, H), jnp.float32),
        compiler_params=pltpu.CompilerParams(
            dimension_semantics=("arbitrary",)),
        interpret=interpret,
    )(te, nt, xs, gate_up_proj, gate_up_proj, down_proj, ws3)


def kernel(hidden_states, topk_weights, topk_indices, gate_up_proj, down_proj):
    T_, K_ = topk_indices.shape
    N = T_ * K_
    tiles = N // M + E
    token_for_slot, ws_slot, invpos, te, nt = _route(
        topk_indices, topk_weights, tiles)
    xs = _sc_gather(hidden_states, token_for_slot)
    ws3 = ws_slot.reshape(tiles, 1, M)
    ys = _tc_gmm(xs, gate_up_proj, down_proj, ws3, te, nt, tiles)
    pos = invpos.reshape(T_, K_)
    out = _sc_combine(ys, pos[:, 0] + 0, pos[:, 1] + 0)
    return out.astype(hidden_states.dtype)


# speedup vs baseline: 1.6975x; 1.6975x over previous
import functools

import jax
import jax.numpy as jnp
from jax import lax
from jax.experimental import pallas as pl
from jax.experimental.pallas import tpu as pltpu
from jax.experimental.pallas import tpu_sc as plsc

NC = 2
NS = 16
NW = NC * NS

E = 16
H = 1024
I = 1024
M = 256


def _route(topk_indices, topk_weights, tiles):
    T_, K_ = topk_indices.shape
    N = T_ * K_
    pad_n = tiles * M
    e_flat = topk_indices.reshape(N).astype(jnp.int32)
    order = jnp.argsort(e_flat, stable=True).astype(jnp.int32)
    sorted_e = e_flat[order]
    g = jnp.zeros((E,), jnp.int32).at[e_flat].add(1)
    seg_start = (jnp.cumsum(g) - g).astype(jnp.int32)
    padded = ((g + (M - 1)) // M) * M
    ends = jnp.cumsum(padded).astype(jnp.int32)
    base = ends - padded
    s = jnp.arange(N, dtype=jnp.int32)
    rank = s - seg_start[sorted_e]
    dest = base[sorted_e] + rank
    tok = order // K_
    pad_fill = jnp.arange(pad_n, dtype=jnp.int32) % T_
    token_for_slot = pad_fill.at[dest].set(tok)
    ws_slot = jnp.zeros((pad_n,), jnp.float32).at[dest].set(
        topk_weights.reshape(N)[order])
    invpos = jnp.zeros((N,), jnp.int32).at[order].set(dest)
    tile_starts = jnp.arange(tiles, dtype=jnp.int32) * M
    te = jnp.searchsorted(ends, tile_starts, side="right").astype(jnp.int32)
    last_e = jnp.searchsorted(ends, ends[-1] - 1, side="right").astype(jnp.int32)
    te = jnp.where(tile_starts < ends[-1], jnp.clip(te, 0, E - 1), last_e)
    nt = (ends[-1:] // M).astype(jnp.int32)
    return token_for_slot, ws_slot, invpos, te, nt


def _sc_gather(x, idx):
    pad_n = idx.shape[0]
    per_w = pad_n // NW
    W = 32
    SUB = 16
    NB = 3
    nch = per_w // W
    mesh = plsc.VectorSubcoreMesh(core_axis_name="core", subcore_axis_name="subcore")

    @functools.partial(
        pl.kernel,
        out_type=jax.ShapeDtypeStruct((pad_n, H), x.dtype),
        mesh=mesh,
        scratch_types=[pltpu.VMEM((2 * nch, SUB), jnp.int32),
                       pltpu.VMEM((W, H), jnp.float32),
                       pltpu.VMEM((W, H), jnp.float32),
                       pltpu.VMEM((W, H), jnp.float32),
                       pltpu.SemaphoreType.DMA,
                       pltpu.SemaphoreType.DMA,
                       pltpu.SemaphoreType.DMA],
    )
    def k(x_hbm, i_hbm, o_hbm, idx_v, r0, r1, r2, s0, s1, s2):
        wid = lax.axis_index("subcore") * NC + lax.axis_index("core")
        base = wid * per_w
        pltpu.sync_copy(i_hbm.at[wid], idx_v)
        bufs = [(r0, s0), (r1, s1), (r2, s2)]

        def fire(c):
            r, s = bufs[c % NB]
            pltpu.async_copy(x_hbm.at[idx_v.at[2 * c]], r.at[pl.ds(0, SUB)], s)
            pltpu.async_copy(x_hbm.at[idx_v.at[2 * c + 1]],
                             r.at[pl.ds(SUB, SUB)], s)

        for c in range(min(NB, nch)):
            fire(c)
        for c in range(nch):
            r, s = bufs[c % NB]
            pltpu.make_async_copy(x_hbm.at[idx_v.at[2 * c]],
                                  r.at[pl.ds(0, SUB)], s).wait()
            pltpu.make_async_copy(x_hbm.at[idx_v.at[2 * c + 1]],
                                  r.at[pl.ds(SUB, SUB)], s).wait()
            pltpu.sync_copy(r, o_hbm.at[pl.ds(base + c * W, W)])
            if c + NB < nch:
                fire(c + NB)

    return k(x, idx.reshape(NW, 2 * nch, SUB))


def _sc_combine(ys, p0, p1):
    T_ = p0.shape[0]
    per_w = T_ // NW
    C = 16
    nch = per_w // C
    mesh = plsc.VectorSubcoreMesh(core_axis_name="core", subcore_axis_name="subcore")

    @functools.partial(
        pl.kernel,
        out_type=jax.ShapeDtypeStruct((T_, H), ys.dtype),
        mesh=mesh,
        scratch_types=[pltpu.VMEM((nch, C), jnp.int32),
                       pltpu.VMEM((nch, C), jnp.int32),
                       pltpu.VMEM((C, H), jnp.float32),
                       pltpu.VMEM((C, H), jnp.float32),
                       pltpu.VMEM((C, H), jnp.float32),
                       pltpu.SemaphoreType.DMA,
                       pltpu.SemaphoreType.DMA],
    )
    def k(ys_hbm, p0_hbm, p1_hbm, o_hbm, p0_v, p1_v, a_v, b_v, o_v, s0, s1):
        wid = lax.axis_index("subcore") * NC + lax.axis_index("core")
        base = wid * per_w
        pltpu.sync_copy(p0_hbm.at[wid], p0_v)
        pltpu.sync_copy(p1_hbm.at[wid], p1_v)
        for c in range(nch):
            pltpu.async_copy(ys_hbm.at[p0_v.at[c]], a_v, s0)
            pltpu.async_copy(ys_hbm.at[p1_v.at[c]], b_v, s1)
            pltpu.make_async_copy(ys_hbm.at[p0_v.at[c]], a_v, s0).wait()
            pltpu.make_async_copy(ys_hbm.at[p1_v.at[c]], b_v, s1).wait()

            @pl.loop(0, C)
            def _(r):
                @pl.loop(0, H, step=16)
                def _(col):
                    o_v[r, pl.ds(col, 16)] = (a_v[r, pl.ds(col, 16)]
                                              + b_v[r, pl.ds(col, 16)])

            pltpu.sync_copy(o_v, o_hbm.at[pl.ds(base + c * C, C)])

    return k(ys, p0.reshape(NW, nch, C), p1.reshape(NW, nch, C))


def _tc_gmm_body(te_ref, nt_ref, xs_ref, g_ref, u_ref, d_ref, ws_ref, ys_ref):
    i = pl.program_id(0)

    @pl.when(i < nt_ref[0])
    def _():
        x = xs_ref[...]
        gate = jnp.dot(x, g_ref[...], preferred_element_type=jnp.float32)
        up = jnp.dot(x, u_ref[...], preferred_element_type=jnp.float32)
        h = (gate * jax.nn.sigmoid(gate)) * up
        y = jnp.dot(h, d_ref[...], preferred_element_type=jnp.float32)
        w = ws_ref[0, 0, :]
        ys_ref[...] = y * w[:, None]


def _tc_gmm(xs, gate_up_proj, down_proj, ws3, te, nt, tiles, interpret=False):
    grid_spec = pltpu.PrefetchScalarGridSpec(
        num_scalar_prefetch=2,
        grid=(tiles,),
        in_specs=[
            pl.BlockSpec((M, H), lambda i, te, nt: (i, 0)),
            pl.BlockSpec((H, I), lambda i, te, nt: (te[i], 0)),
            pl.BlockSpec((H, I), lambda i, te, nt: (te[i], 1)),
            pl.BlockSpec((I, H), lambda i, te, nt: (te[i], 0)),
            pl.BlockSpec((1, 1, M), lambda i, te, nt: (i, 0, 0)),
        ],
        out_specs=pl.BlockSpec((M, H), lambda i, te, nt: (i, 0)),
    )
    return pl.pallas_call(
        _tc_gmm_body,
        grid_spec=grid_spec,
        out_shape=jax.ShapeDtypeStruct((tiles * M, H), jnp.float32),
        compiler_params=pltpu.CompilerParams(
            dimension_semantics=("arbitrary",)),
        interpret=interpret,
    )(te, nt, xs, gate_up_proj, gate_up_proj, down_proj, ws3)


def kernel(hidden_states, topk_weights, topk_indices, gate_up_proj, down_proj):
    T_, K_ = topk_indices.shape
    N = T_ * K_
    tiles = N // M + E
    token_for_slot, ws_slot, invpos, te, nt = _route(
        topk_indices, topk_weights, tiles)
    xs = _sc_gather(hidden_states, token_for_slot)
    ws3 = ws_slot.reshape(tiles, 1, M)
    ys = _tc_gmm(xs, gate_up_proj, down_proj, ws3, te, nt, tiles)
    pos = invpos.reshape(T_, K_)
    out = _sc_combine(ys, pos[:, 0] + 0, pos[:, 1] + 0)
    return out.astype(hidden_states.dtype)


# speedup vs baseline: 2.2092x; 1.3014x over previous
import functools

import jax
import jax.numpy as jnp
from jax import lax
from jax.experimental import pallas as pl
from jax.experimental.pallas import tpu as pltpu
from jax.experimental.pallas import tpu_sc as plsc

NC = 2
NS = 16
NW = NC * NS

E = 16
H = 1024
I = 1024
M = 256


def _route(topk_indices, tiles):
    T_, K_ = topk_indices.shape
    N = T_ * K_
    e_flat = topk_indices.reshape(N).astype(jnp.int32)
    oneh = (e_flat[:, None] == jnp.arange(E, dtype=jnp.int32)[None, :])
    counts = jnp.cumsum(oneh.astype(jnp.int32), axis=0)
    g = counts[-1]
    padded = ((g + (M - 1)) // M) * M
    ends = jnp.cumsum(padded).astype(jnp.int32)
    base = ends - padded
    rank = jnp.take_along_axis(counts, e_flat[:, None], axis=1)[:, 0] - 1
    invpos = (base[e_flat] + rank).astype(jnp.int32)
    tile_starts = jnp.arange(tiles, dtype=jnp.int32) * M
    te = jnp.searchsorted(ends, tile_starts, side="right").astype(jnp.int32)
    last_e = jnp.searchsorted(ends, ends[-1] - 1, side="right").astype(jnp.int32)
    te = jnp.where(tile_starts < ends[-1], jnp.clip(te, 0, E - 1), last_e)
    nt = (ends[-1:] // M).astype(jnp.int32)
    return invpos, te, nt


def _sc_disperse(x, p0, p1, pad_n):
    T_ = p0.shape[0]
    per_w = T_ // NW
    mesh = plsc.VectorSubcoreMesh(core_axis_name="core", subcore_axis_name="subcore")

    @functools.partial(
        pl.kernel,
        out_type=jax.ShapeDtypeStruct((pad_n, H), x.dtype),
        mesh=mesh,
        scratch_types=[pltpu.VMEM((per_w,), jnp.int32),
                       pltpu.VMEM((per_w,), jnp.int32),
                       pltpu.VMEM((per_w, H), jnp.float32),
                       pltpu.SemaphoreType.DMA,
                       pltpu.SemaphoreType.DMA],
    )
    def k(x_hbm, p0_hbm, p1_hbm, o_hbm, p0_v, p1_v, rows_v, s0, s1):
        wid = lax.axis_index("subcore") * NC + lax.axis_index("core")
        base = wid * per_w
        pltpu.sync_copy(p0_hbm.at[pl.ds(base, per_w)], p0_v)
        pltpu.sync_copy(p1_hbm.at[pl.ds(base, per_w)], p1_v)
        pltpu.sync_copy(x_hbm.at[pl.ds(base, per_w)], rows_v)
        pltpu.async_copy(rows_v, o_hbm.at[p0_v], s0)
        pltpu.async_copy(rows_v, o_hbm.at[p1_v], s1)
        pltpu.make_async_copy(rows_v, o_hbm.at[p0_v], s0).wait()
        pltpu.make_async_copy(rows_v, o_hbm.at[p1_v], s1).wait()

    return k(x, p0, p1)


def _sc_combine(ys, p0, p1, w0, w1):
    T_ = p0.shape[0]
    per_w = T_ // NW
    C = 16
    nch = per_w // C
    L = 16
    mesh = plsc.VectorSubcoreMesh(core_axis_name="core", subcore_axis_name="subcore")

    @functools.partial(
        pl.kernel,
        out_type=jax.ShapeDtypeStruct((T_, H), ys.dtype),
        mesh=mesh,
        scratch_types=[pltpu.VMEM((nch, C), jnp.int32),
                       pltpu.VMEM((nch, C), jnp.int32),
                       pltpu.VMEM((per_w, 16), jnp.float32),
                       pltpu.VMEM((per_w, 16), jnp.float32),
                       pltpu.VMEM((C, H), jnp.float32),
                       pltpu.VMEM((C, H), jnp.float32),
                       pltpu.VMEM((C, H), jnp.float32),
                       pltpu.SemaphoreType.DMA,
                       pltpu.SemaphoreType.DMA],
    )
    def k(ys_hbm, p0_hbm, p1_hbm, w0_hbm, w1_hbm, o_hbm,
          p0_v, p1_v, w0_v, w1_v, a_v, b_v, o_v, s0, s1):
        wid = lax.axis_index("subcore") * NC + lax.axis_index("core")
        base = wid * per_w
        pltpu.sync_copy(p0_hbm.at[wid], p0_v)
        pltpu.sync_copy(p1_hbm.at[wid], p1_v)
        pltpu.sync_copy(w0_hbm.at[pl.ds(base, per_w)], w0_v)
        pltpu.sync_copy(w1_hbm.at[pl.ds(base, per_w)], w1_v)
        for c in range(nch):
            pltpu.async_copy(ys_hbm.at[p0_v.at[c]], a_v, s0)
            pltpu.async_copy(ys_hbm.at[p1_v.at[c]], b_v, s1)
            pltpu.make_async_copy(ys_hbm.at[p0_v.at[c]], a_v, s0).wait()
            pltpu.make_async_copy(ys_hbm.at[p1_v.at[c]], b_v, s1).wait()

            @pl.loop(0, C)
            def _(r):
                wa = w0_v[c * C + r, :]
                wb = w1_v[c * C + r, :]

                @pl.loop(0, H, step=L)
                def _(col):
                    o_v[r, pl.ds(col, L)] = (a_v[r, pl.ds(col, L)] * wa
                                             + b_v[r, pl.ds(col, L)] * wb)

            pltpu.sync_copy(o_v, o_hbm.at[pl.ds(base + c * C, C)])

    return k(ys, p0.reshape(NW, nch, C), p1.reshape(NW, nch, C), w0, w1)


def _tc_gmm_body(te_ref, nt_ref, xs_ref, g_ref, u_ref, d_ref, ys_ref):
    i = pl.program_id(0)

    @pl.when(i < nt_ref[0])
    def _():
        x = xs_ref[...]
        gate = jnp.dot(x, g_ref[...], preferred_element_type=jnp.float32)
        up = jnp.dot(x, u_ref[...], preferred_element_type=jnp.float32)
        h = (gate * jax.nn.sigmoid(gate)) * up
        ys_ref[...] = jnp.dot(h, d_ref[...], preferred_element_type=jnp.float32)


def _tc_gmm(xs, gate_up_proj, down_proj, te, nt, tiles, interpret=False):
    grid_spec = pltpu.PrefetchScalarGridSpec(
        num_scalar_prefetch=2,
        grid=(tiles,),
        in_specs=[
            pl.BlockSpec((M, H), lambda i, te, nt: (i, 0)),
            pl.BlockSpec((H, I), lambda i, te, nt: (te[i], 0)),
            pl.BlockSpec((H, I), lambda i, te, nt: (te[i], 1)),
            pl.BlockSpec((I, H), lambda i, te, nt: (te[i], 0)),
        ],
        out_specs=pl.BlockSpec((M, H), lambda i, te, nt: (i, 0)),
    )
    return pl.pallas_call(
        _tc_gmm_body,
        grid_spec=grid_spec,
        out_shape=jax.ShapeDtypeStruct((tiles * M, H), jnp.float32),
        compiler_params=pltpu.CompilerParams(
            dimension_semantics=("arbitrary",)),
        interpret=interpret,
    )(te, nt, xs, gate_up_proj, gate_up_proj, down_proj)


def kernel(hidden_states, topk_weights, topk_indices, gate_up_proj, down_proj):
    T_, K_ = topk_indices.shape
    N = T_ * K_
    tiles = N // M + E
    invpos, te, nt = _route(topk_indices, tiles)
    pos = invpos.reshape(T_, K_)
    p0 = pos[:, 0] + 0
    p1 = pos[:, 1] + 0
    w0 = jnp.broadcast_to(topk_weights[:, 0:1], (T_, 16)) + 0.0
    w1 = jnp.broadcast_to(topk_weights[:, 1:2], (T_, 16)) + 0.0
    xs = _sc_disperse(hidden_states, p0, p1, tiles * M)
    ys = _tc_gmm(xs, gate_up_proj, down_proj, te, nt, tiles)
    out = _sc_combine(ys, p0, p1, w0, w1)
    return out.astype(hidden_states.dtype)


# speedup vs baseline: 2.4572x; 1.1122x over previous
import functools

import jax
import jax.numpy as jnp
from jax import lax
from jax.experimental import pallas as pl
from jax.experimental.pallas import tpu as pltpu
from jax.experimental.pallas import tpu_sc as plsc

NC = 2
NS = 16
NW = NC * NS

E = 16
H = 1024
I = 1024
M = 256


def _route(topk_indices, tiles):
    T_, K_ = topk_indices.shape
    N = T_ * K_
    e_flat = topk_indices.reshape(N).astype(jnp.int32)
    oneh = (e_flat[:, None] == jnp.arange(E, dtype=jnp.int32)[None, :])
    oneh_i = oneh.astype(jnp.int32)
    counts = jnp.cumsum(oneh_i, axis=0)
    g = counts[-1]
    padded = ((g + (M - 1)) // M) * M
    ends = jnp.cumsum(padded).astype(jnp.int32)
    base = ends - padded
    rank = jnp.sum(jnp.where(oneh, counts, 0), axis=1) - 1
    slot_base = jnp.sum(jnp.where(oneh, base[None, :], 0), axis=1)
    invpos = (slot_base + rank).astype(jnp.int32)
    tile_starts = jnp.arange(tiles, dtype=jnp.int32) * M
    te = jnp.sum((ends[None, :] <= tile_starts[:, None]).astype(jnp.int32),
                 axis=1)
    last_e = jnp.sum((ends <= ends[-1] - 1).astype(jnp.int32))
    te = jnp.where(tile_starts < ends[-1], jnp.clip(te, 0, E - 1), last_e)
    nt = (ends[-1:] // M).astype(jnp.int32)
    return invpos, te.astype(jnp.int32), nt


def _sc_disperse(x, p0, p1, pad_n):
    T_ = p0.shape[0]
    per_w = T_ // NW
    mesh = plsc.VectorSubcoreMesh(core_axis_name="core", subcore_axis_name="subcore")

    @functools.partial(
        pl.kernel,
        out_type=jax.ShapeDtypeStruct((pad_n, H), x.dtype),
        mesh=mesh,
        scratch_types=[pltpu.VMEM((per_w,), jnp.int32),
                       pltpu.VMEM((per_w,), jnp.int32),
                       pltpu.VMEM((per_w, H), jnp.float32),
                       pltpu.SemaphoreType.DMA,
                       pltpu.SemaphoreType.DMA],
    )
    def k(x_hbm, p0_hbm, p1_hbm, o_hbm, p0_v, p1_v, rows_v, s0, s1):
        wid = lax.axis_index("subcore") * NC + lax.axis_index("core")
        base = wid * per_w
        pltpu.sync_copy(p0_hbm.at[pl.ds(base, per_w)], p0_v)
        pltpu.sync_copy(p1_hbm.at[pl.ds(base, per_w)], p1_v)
        pltpu.sync_copy(x_hbm.at[pl.ds(base, per_w)], rows_v)
        pltpu.async_copy(rows_v, o_hbm.at[p0_v], s0)
        pltpu.async_copy(rows_v, o_hbm.at[p1_v], s1)
        pltpu.make_async_copy(rows_v, o_hbm.at[p0_v], s0).wait()
        pltpu.make_async_copy(rows_v, o_hbm.at[p1_v], s1).wait()

    return k(x, p0, p1)


def _sc_combine(ys, p0, p1, w0, w1):
    T_ = p0.shape[0]
    per_w = T_ // NW
    C = 16
    nch = per_w // C
    L = 16
    mesh = plsc.VectorSubcoreMesh(core_axis_name="core", subcore_axis_name="subcore")

    @functools.partial(
        pl.kernel,
        out_type=jax.ShapeDtypeStruct((T_, H), ys.dtype),
        mesh=mesh,
        scratch_types=[pltpu.VMEM((nch, C), jnp.int32),
                       pltpu.VMEM((nch, C), jnp.int32),
                       pltpu.VMEM((per_w, 16), jnp.float32),
                       pltpu.VMEM((per_w, 16), jnp.float32),
                       pltpu.VMEM((2, C, H), jnp.float32),
                       pltpu.VMEM((2, C, H), jnp.float32),
                       pltpu.VMEM((C, H), jnp.float32),
                       pltpu.SemaphoreType.DMA,
                       pltpu.SemaphoreType.DMA,
                       pltpu.SemaphoreType.DMA,
                       pltpu.SemaphoreType.DMA],
    )
    def k(ys_hbm, p0_hbm, p1_hbm, w0_hbm, w1_hbm, o_hbm,
          p0_v, p1_v, w0_v, w1_v, a_v, b_v, o_v, s0, s1, s2, s3):
        wid = lax.axis_index("subcore") * NC + lax.axis_index("core")
        base = wid * per_w
        pltpu.sync_copy(p0_hbm.at[wid], p0_v)
        pltpu.sync_copy(p1_hbm.at[wid], p1_v)
        pltpu.sync_copy(w0_hbm.at[pl.ds(base, per_w)], w0_v)
        pltpu.sync_copy(w1_hbm.at[pl.ds(base, per_w)], w1_v)
        sems = [(s0, s1), (s2, s3)]

        def fire(c):
            sa, sb = sems[c % 2]
            pltpu.async_copy(ys_hbm.at[p0_v.at[c]], a_v.at[c % 2], sa)
            pltpu.async_copy(ys_hbm.at[p1_v.at[c]], b_v.at[c % 2], sb)

        fire(0)
        for c in range(nch):
            if c + 1 < nch:
                fire(c + 1)
            sa, sb = sems[c % 2]
            pltpu.make_async_copy(ys_hbm.at[p0_v.at[c]], a_v.at[c % 2], sa).wait()
            pltpu.make_async_copy(ys_hbm.at[p1_v.at[c]], b_v.at[c % 2], sb).wait()

            @pl.loop(0, C)
            def _(r):
                wa = w0_v[c * C + r, :]
                wb = w1_v[c * C + r, :]

                @pl.loop(0, H, step=L)
                def _(col):
                    o_v[r, pl.ds(col, L)] = (
                        a_v[c % 2, r, pl.ds(col, L)] * wa
                        + b_v[c % 2, r, pl.ds(col, L)] * wb)

            pltpu.sync_copy(o_v, o_hbm.at[pl.ds(base + c * C, C)])

    return k(ys, p0.reshape(NW, nch, C), p1.reshape(NW, nch, C), w0, w1)


def _tc_gmm_body(te_ref, nt_ref, xs_ref, g_ref, u_ref, d_ref, ys_ref):
    i = pl.program_id(0)

    @pl.when(i < nt_ref[0])
    def _():
        x = xs_ref[...]
        gate = jnp.dot(x, g_ref[...], preferred_element_type=jnp.float32)
        up = jnp.dot(x, u_ref[...], preferred_element_type=jnp.float32)
        h = (gate * jax.nn.sigmoid(gate)) * up
        ys_ref[...] = jnp.dot(h, d_ref[...], preferred_element_type=jnp.float32)


def _tc_gmm(xs, gate_up_proj, down_proj, te, nt, tiles, interpret=False):
    grid_spec = pltpu.PrefetchScalarGridSpec(
        num_scalar_prefetch=2,
        grid=(tiles,),
        in_specs=[
            pl.BlockSpec((M, H), lambda i, te, nt: (i, 0)),
            pl.BlockSpec((H, I), lambda i, te, nt: (te[i], 0)),
            pl.BlockSpec((H, I), lambda i, te, nt: (te[i], 1)),
            pl.BlockSpec((I, H), lambda i, te, nt: (te[i], 0)),
        ],
        out_specs=pl.BlockSpec((M, H), lambda i, te, nt: (i, 0)),
    )
    return pl.pallas_call(
        _tc_gmm_body,
        grid_spec=grid_spec,
        out_shape=jax.ShapeDtypeStruct((tiles * M, H), jnp.float32),
        compiler_params=pltpu.CompilerParams(
            dimension_semantics=("arbitrary",)),
        interpret=interpret,
    )(te, nt, xs, gate_up_proj, gate_up_proj, down_proj)


def kernel(hidden_states, topk_weights, topk_indices, gate_up_proj, down_proj):
    T_, K_ = topk_indices.shape
    N = T_ * K_
    tiles = N // M + E
    invpos, te, nt = _route(topk_indices, tiles)
    pos = invpos.reshape(T_, K_)
    p0 = pos[:, 0] + 0
    p1 = pos[:, 1] + 0
    w0 = jnp.broadcast_to(topk_weights[:, 0:1], (T_, 16)) + 0.0
    w1 = jnp.broadcast_to(topk_weights[:, 1:2], (T_, 16)) + 0.0
    xs = _sc_disperse(hidden_states, p0, p1, tiles * M)
    ys = _tc_gmm(xs, gate_up_proj, down_proj, te, nt, tiles)
    out = _sc_combine(ys, p0, p1, w0, w1)
    return out.astype(hidden_states.dtype)


# speedup vs baseline: 2.5570x; 1.0406x over previous
import functools

import jax
import jax.numpy as jnp
from jax import lax
from jax.experimental import pallas as pl
from jax.experimental.pallas import tpu as pltpu
from jax.experimental.pallas import tpu_sc as plsc

NC = 2
NS = 16
NW = NC * NS

E = 16
H = 1024
I = 1024
M = 256


def _route(topk_indices, tiles):
    T_, K_ = topk_indices.shape
    N = T_ * K_
    e_flat = topk_indices.reshape(N).astype(jnp.int32)
    oneh = (e_flat[:, None] == jnp.arange(E, dtype=jnp.int32)[None, :])
    oneh_i = oneh.astype(jnp.int32)
    counts = jnp.cumsum(oneh_i, axis=0)
    g = counts[-1]
    padded = ((g + (M - 1)) // M) * M
    ends = jnp.cumsum(padded).astype(jnp.int32)
    base = ends - padded
    rank = jnp.sum(jnp.where(oneh, counts, 0), axis=1) - 1
    slot_base = jnp.sum(jnp.where(oneh, base[None, :], 0), axis=1)
    invpos = (slot_base + rank).astype(jnp.int32)
    tile_starts = jnp.arange(tiles, dtype=jnp.int32) * M
    te = jnp.sum((ends[None, :] <= tile_starts[:, None]).astype(jnp.int32),
                 axis=1)
    last_e = jnp.sum((ends <= ends[-1] - 1).astype(jnp.int32))
    te = jnp.where(tile_starts < ends[-1], jnp.clip(te, 0, E - 1), last_e)
    nt = (ends[-1:] // M).astype(jnp.int32)
    return invpos, te.astype(jnp.int32), nt


def _sc_disperse(x, p0, p1, pad_n):
    T_ = p0.shape[0]
    per_w = T_ // NW
    mesh = plsc.VectorSubcoreMesh(core_axis_name="core", subcore_axis_name="subcore")

    @functools.partial(
        pl.kernel,
        out_type=jax.ShapeDtypeStruct((pad_n, H), x.dtype),
        mesh=mesh,
        scratch_types=[pltpu.VMEM((per_w,), jnp.int32),
                       pltpu.VMEM((per_w,), jnp.int32),
                       pltpu.VMEM((per_w, H), jnp.float32),
                       pltpu.SemaphoreType.DMA,
                       pltpu.SemaphoreType.DMA],
    )
    def k(x_hbm, p0_hbm, p1_hbm, o_hbm, p0_v, p1_v, rows_v, s0, s1):
        wid = lax.axis_index("subcore") * NC + lax.axis_index("core")
        base = wid * per_w
        pltpu.sync_copy(p0_hbm.at[pl.ds(base, per_w)], p0_v)
        pltpu.sync_copy(p1_hbm.at[pl.ds(base, per_w)], p1_v)
        pltpu.sync_copy(x_hbm.at[pl.ds(base, per_w)], rows_v)
        pltpu.async_copy(rows_v, o_hbm.at[p0_v], s0)
        pltpu.async_copy(rows_v, o_hbm.at[p1_v], s1)
        pltpu.make_async_copy(rows_v, o_hbm.at[p0_v], s0).wait()
        pltpu.make_async_copy(rows_v, o_hbm.at[p1_v], s1).wait()

    return k(x, p0, p1)


def _sc_combine(ys, p0, p1, w0, w1):
    T_ = p0.shape[0]
    per_w = T_ // NW
    C = 16
    nch = per_w // C
    L = 16
    mesh = plsc.VectorSubcoreMesh(core_axis_name="core", subcore_axis_name="subcore")

    @functools.partial(
        pl.kernel,
        out_type=jax.ShapeDtypeStruct((T_, H), ys.dtype),
        mesh=mesh,
        scratch_types=[pltpu.VMEM((nch, C), jnp.int32),
                       pltpu.VMEM((nch, C), jnp.int32),
                       pltpu.VMEM((per_w, 16), jnp.float32),
                       pltpu.VMEM((per_w, 16), jnp.float32),
                       pltpu.VMEM((2, C, H), jnp.float32),
                       pltpu.VMEM((2, C, H), jnp.float32),
                       pltpu.VMEM((C, H), jnp.float32),
                       pltpu.SemaphoreType.DMA,
                       pltpu.SemaphoreType.DMA,
                       pltpu.SemaphoreType.DMA,
                       pltpu.SemaphoreType.DMA],
    )
    def k(ys_hbm, p0_hbm, p1_hbm, w0_hbm, w1_hbm, o_hbm,
          p0_v, p1_v, w0_v, w1_v, a_v, b_v, o_v, s0, s1, s2, s3):
        wid = lax.axis_index("subcore") * NC + lax.axis_index("core")
        base = wid * per_w
        pltpu.sync_copy(p0_hbm.at[wid], p0_v)
        pltpu.sync_copy(p1_hbm.at[wid], p1_v)
        pltpu.sync_copy(w0_hbm.at[pl.ds(base, per_w)], w0_v)
        pltpu.sync_copy(w1_hbm.at[pl.ds(base, per_w)], w1_v)
        sems = [(s0, s1), (s2, s3)]

        def fire(c):
            sa, sb = sems[c % 2]
            pltpu.async_copy(ys_hbm.at[p0_v.at[c]], a_v.at[c % 2], sa)
            pltpu.async_copy(ys_hbm.at[p1_v.at[c]], b_v.at[c % 2], sb)

        fire(0)
        for c in range(nch):
            if c + 1 < nch:
                fire(c + 1)
            sa, sb = sems[c % 2]
            pltpu.make_async_copy(ys_hbm.at[p0_v.at[c]], a_v.at[c % 2], sa).wait()
            pltpu.make_async_copy(ys_hbm.at[p1_v.at[c]], b_v.at[c % 2], sb).wait()

            @pl.loop(0, C)
            def _(r):
                wa = w0_v[c * C + r, :]
                wb = w1_v[c * C + r, :]

                @pl.loop(0, H, step=L)
                def _(col):
                    o_v[r, pl.ds(col, L)] = (
                        a_v[c % 2, r, pl.ds(col, L)] * wa
                        + b_v[c % 2, r, pl.ds(col, L)] * wb)

            pltpu.sync_copy(o_v, o_hbm.at[pl.ds(base + c * C, C)])

    return k(ys, p0.reshape(NW, nch, C), p1.reshape(NW, nch, C), w0, w1)


def _tc_gmm_body(te_ref, nt_ref, xs_ref, g_ref, u_ref, d_ref, ys_ref):
    i = pl.program_id(0)

    @pl.when(i < nt_ref[0])
    def _():
        x = xs_ref[...]
        gate = jnp.dot(x, g_ref[...], preferred_element_type=jnp.float32)
        up = jnp.dot(x, u_ref[...], preferred_element_type=jnp.float32)
        h = (gate * jax.nn.sigmoid(gate)) * up
        ys_ref[...] = jnp.dot(h, d_ref[...], preferred_element_type=jnp.float32)


def _tc_gmm(xs, gate_up_proj, down_proj, te, nt, tiles, interpret=False):
    grid_spec = pltpu.PrefetchScalarGridSpec(
        num_scalar_prefetch=2,
        grid=(tiles,),
        in_specs=[
            pl.BlockSpec((M, H),
                         lambda i, te, nt: (jnp.minimum(i, nt[0] - 1), 0)),
            pl.BlockSpec((H, I), lambda i, te, nt: (te[i], 0)),
            pl.BlockSpec((H, I), lambda i, te, nt: (te[i], 1)),
            pl.BlockSpec((I, H), lambda i, te, nt: (te[i], 0)),
        ],
        out_specs=pl.BlockSpec(
            (M, H), lambda i, te, nt: (jnp.minimum(i, nt[0] - 1), 0)),
    )
    return pl.pallas_call(
        _tc_gmm_body,
        grid_spec=grid_spec,
        out_shape=jax.ShapeDtypeStruct((tiles * M, H), jnp.float32),
        compiler_params=pltpu.CompilerParams(
            dimension_semantics=("arbitrary",)),
        interpret=interpret,
    )(te, nt, xs, gate_up_proj, gate_up_proj, down_proj)


def kernel(hidden_states, topk_weights, topk_indices, gate_up_proj, down_proj):
    T_, K_ = topk_indices.shape
    N = T_ * K_
    tiles = N // M + E
    invpos, te, nt = _route(topk_indices, tiles)
    pos = invpos.reshape(T_, K_)
    p0 = pos[:, 0] + 0
    p1 = pos[:, 1] + 0
    w0 = jnp.broadcast_to(topk_weights[:, 0:1], (T_, 16)) + 0.0
    w1 = jnp.broadcast_to(topk_weights[:, 1:2], (T_, 16)) + 0.0
    xs = _sc_disperse(hidden_states, p0, p1, tiles * M)
    ys = _tc_gmm(xs, gate_up_proj, down_proj, te, nt, tiles)
    out = _sc_combine(ys, p0, p1, w0, w1)
    return out.astype(hidden_states.dtype)


# speedup vs baseline: 2.7262x; 1.0662x over previous
import functools

import jax
import jax.numpy as jnp
from jax import lax
from jax.experimental import pallas as pl
from jax.experimental.pallas import tpu as pltpu
from jax.experimental.pallas import tpu_sc as plsc

NC = 2
NS = 16
NW = NC * NS

E = 16
H = 1024
I = 1024
M = 512


def _route(topk_indices, tiles):
    T_, K_ = topk_indices.shape
    N = T_ * K_
    e_flat = topk_indices.reshape(N).astype(jnp.int32)
    oneh = (e_flat[:, None] == jnp.arange(E, dtype=jnp.int32)[None, :])
    oneh_i = oneh.astype(jnp.int32)
    counts = jnp.cumsum(oneh_i, axis=0)
    g = counts[-1]
    padded = ((g + (M - 1)) // M) * M
    ends = jnp.cumsum(padded).astype(jnp.int32)
    base = ends - padded
    rank = jnp.sum(jnp.where(oneh, counts, 0), axis=1) - 1
    slot_base = jnp.sum(jnp.where(oneh, base[None, :], 0), axis=1)
    invpos = (slot_base + rank).astype(jnp.int32)
    tile_starts = jnp.arange(tiles, dtype=jnp.int32) * M
    te = jnp.sum((ends[None, :] <= tile_starts[:, None]).astype(jnp.int32),
                 axis=1)
    last_e = jnp.sum((ends <= ends[-1] - 1).astype(jnp.int32))
    te = jnp.where(tile_starts < ends[-1], jnp.clip(te, 0, E - 1), last_e)
    nt = (ends[-1:] // M).astype(jnp.int32)
    return invpos, te.astype(jnp.int32), nt


def _sc_disperse(x, p0, p1, pad_n):
    T_ = p0.shape[0]
    per_w = T_ // NW
    mesh = plsc.VectorSubcoreMesh(core_axis_name="core", subcore_axis_name="subcore")

    @functools.partial(
        pl.kernel,
        out_type=jax.ShapeDtypeStruct((pad_n, H), x.dtype),
        mesh=mesh,
        scratch_types=[pltpu.VMEM((per_w,), jnp.int32),
                       pltpu.VMEM((per_w,), jnp.int32),
                       pltpu.VMEM((per_w, H), jnp.float32),
                       pltpu.SemaphoreType.DMA,
                       pltpu.SemaphoreType.DMA],
    )
    def k(x_hbm, p0_hbm, p1_hbm, o_hbm, p0_v, p1_v, rows_v, s0, s1):
        wid = lax.axis_index("subcore") * NC + lax.axis_index("core")
        base = wid * per_w
        pltpu.sync_copy(p0_hbm.at[pl.ds(base, per_w)], p0_v)
        pltpu.sync_copy(p1_hbm.at[pl.ds(base, per_w)], p1_v)
        pltpu.sync_copy(x_hbm.at[pl.ds(base, per_w)], rows_v)
        pltpu.async_copy(rows_v, o_hbm.at[p0_v], s0)
        pltpu.async_copy(rows_v, o_hbm.at[p1_v], s1)
        pltpu.make_async_copy(rows_v, o_hbm.at[p0_v], s0).wait()
        pltpu.make_async_copy(rows_v, o_hbm.at[p1_v], s1).wait()

    return k(x, p0, p1)


def _sc_combine(ys, p0, p1, w0, w1):
    T_ = p0.shape[0]
    per_w = T_ // NW
    C = 16
    nch = per_w // C
    L = 16
    mesh = plsc.VectorSubcoreMesh(core_axis_name="core", subcore_axis_name="subcore")

    @functools.partial(
        pl.kernel,
        out_type=jax.ShapeDtypeStruct((T_, H), ys.dtype),
        mesh=mesh,
        scratch_types=[pltpu.VMEM((nch, C), jnp.int32),
                       pltpu.VMEM((nch, C), jnp.int32),
                       pltpu.VMEM((per_w, 16), jnp.float32),
                       pltpu.VMEM((per_w, 16), jnp.float32),
                       pltpu.VMEM((2, C, H), jnp.float32),
                       pltpu.VMEM((2, C, H), jnp.float32),
                       pltpu.VMEM((C, H), jnp.float32),
                       pltpu.SemaphoreType.DMA,
                       pltpu.SemaphoreType.DMA,
                       pltpu.SemaphoreType.DMA,
                       pltpu.SemaphoreType.DMA],
    )
    def k(ys_hbm, p0_hbm, p1_hbm, w0_hbm, w1_hbm, o_hbm,
          p0_v, p1_v, w0_v, w1_v, a_v, b_v, o_v, s0, s1, s2, s3):
        wid = lax.axis_index("subcore") * NC + lax.axis_index("core")
        base = wid * per_w
        pltpu.sync_copy(p0_hbm.at[wid], p0_v)
        pltpu.sync_copy(p1_hbm.at[wid], p1_v)
        pltpu.sync_copy(w0_hbm.at[pl.ds(base, per_w)], w0_v)
        pltpu.sync_copy(w1_hbm.at[pl.ds(base, per_w)], w1_v)
        sems = [(s0, s1), (s2, s3)]

        def fire(c):
            sa, sb = sems[c % 2]
            pltpu.async_copy(ys_hbm.at[p0_v.at[c]], a_v.at[c % 2], sa)
            pltpu.async_copy(ys_hbm.at[p1_v.at[c]], b_v.at[c % 2], sb)

        fire(0)
        for c in range(nch):
            if c + 1 < nch:
                fire(c + 1)
            sa, sb = sems[c % 2]
            pltpu.make_async_copy(ys_hbm.at[p0_v.at[c]], a_v.at[c % 2], sa).wait()
            pltpu.make_async_copy(ys_hbm.at[p1_v.at[c]], b_v.at[c % 2], sb).wait()

            @pl.loop(0, C)
            def _(r):
                wa = w0_v[c * C + r, :]
                wb = w1_v[c * C + r, :]

                @pl.loop(0, H, step=L)
                def _(col):
                    o_v[r, pl.ds(col, L)] = (
                        a_v[c % 2, r, pl.ds(col, L)] * wa
                        + b_v[c % 2, r, pl.ds(col, L)] * wb)

            pltpu.sync_copy(o_v, o_hbm.at[pl.ds(base + c * C, C)])

    return k(ys, p0.reshape(NW, nch, C), p1.reshape(NW, nch, C), w0, w1)


def _tc_gmm_body(te_ref, nt_ref, xs_ref, g_ref, u_ref, d_ref, ys_ref):
    i = pl.program_id(0)

    @pl.when(i < nt_ref[0])
    def _():
        x = xs_ref[...]
        gate = jnp.dot(x, g_ref[...], preferred_element_type=jnp.float32)
        up = jnp.dot(x, u_ref[...], preferred_element_type=jnp.float32)
        h = (gate * jax.nn.sigmoid(gate)) * up
        ys_ref[...] = jnp.dot(h, d_ref[...], preferred_element_type=jnp.float32)


def _tc_gmm(xs, gate_up_proj, down_proj, te, nt, tiles, interpret=False):
    grid_spec = pltpu.PrefetchScalarGridSpec(
        num_scalar_prefetch=2,
        grid=(tiles,),
        in_specs=[
            pl.BlockSpec((M, H),
                         lambda i, te, nt: (jnp.minimum(i, nt[0] - 1), 0)),
            pl.BlockSpec((H, I), lambda i, te, nt: (te[i], 0)),
            pl.BlockSpec((H, I), lambda i, te, nt: (te[i], 1)),
            pl.BlockSpec((I, H), lambda i, te, nt: (te[i], 0)),
        ],
        out_specs=pl.BlockSpec(
            (M, H), lambda i, te, nt: (jnp.minimum(i, nt[0] - 1), 0)),
    )
    return pl.pallas_call(
        _tc_gmm_body,
        grid_spec=grid_spec,
        out_shape=jax.ShapeDtypeStruct((tiles * M, H), jnp.float32),
        compiler_params=pltpu.CompilerParams(
            dimension_semantics=("arbitrary",)),
        interpret=interpret,
    )(te, nt, xs, gate_up_proj, gate_up_proj, down_proj)


def kernel(hidden_states, topk_weights, topk_indices, gate_up_proj, down_proj):
    T_, K_ = topk_indices.shape
    N = T_ * K_
    tiles = N // M + E
    invpos, te, nt = _route(topk_indices, tiles)
    pos = invpos.reshape(T_, K_)
    p0 = pos[:, 0] + 0
    p1 = pos[:, 1] + 0
    w0 = jnp.broadcast_to(topk_weights[:, 0:1], (T_, 16)) + 0.0
    w1 = jnp.broadcast_to(topk_weights[:, 1:2], (T_, 16)) + 0.0
    xs = _sc_disperse(hidden_states, p0, p1, tiles * M)
    ys = _tc_gmm(xs, gate_up_proj, down_proj, te, nt, tiles)
    out = _sc_combine(ys, p0, p1, w0, w1)
    return out.astype(hidden_states.dtype)


# speedup vs baseline: 2.7308x; 1.0017x over previous
import functools

import jax
import jax.numpy as jnp
from jax import lax
from jax.experimental import pallas as pl
from jax.experimental.pallas import tpu as pltpu
from jax.experimental.pallas import tpu_sc as plsc

NC = 2
NS = 16
NW = NC * NS

E = 16
H = 1024
I = 1024
M = 512


def _route(topk_indices, tiles):
    T_, K_ = topk_indices.shape
    N = T_ * K_
    e_flat = topk_indices.reshape(N).astype(jnp.int32)
    oneh = (e_flat[:, None] == jnp.arange(E, dtype=jnp.int32)[None, :])
    counts = jnp.cumsum(oneh.astype(jnp.int16), axis=0).astype(jnp.int32)
    g = counts[-1]
    padded = ((g + (M - 1)) // M) * M
    ends = jnp.cumsum(padded).astype(jnp.int32)
    base = ends - padded
    rank = jnp.sum(jnp.where(oneh, counts, 0), axis=1) - 1
    slot_base = jnp.sum(jnp.where(oneh, base[None, :], 0), axis=1)
    invpos = (slot_base + rank).astype(jnp.int32)
    tile_starts = jnp.arange(tiles, dtype=jnp.int32) * M
    te = jnp.sum((ends[None, :] <= tile_starts[:, None]).astype(jnp.int32),
                 axis=1)
    last_e = jnp.sum((ends <= ends[-1] - 1).astype(jnp.int32))
    te = jnp.where(tile_starts < ends[-1], jnp.clip(te, 0, E - 1), last_e)
    nt = (ends[-1:] // M).astype(jnp.int32)
    return invpos, te.astype(jnp.int32), nt


def _sc_disperse(x, p0, p1, pad_n):
    T_ = p0.shape[0]
    per_w = T_ // NW
    mesh = plsc.VectorSubcoreMesh(core_axis_name="core", subcore_axis_name="subcore")

    @functools.partial(
        pl.kernel,
        out_type=jax.ShapeDtypeStruct((pad_n, H), x.dtype),
        mesh=mesh,
        scratch_types=[pltpu.VMEM((per_w,), jnp.int32),
                       pltpu.VMEM((per_w,), jnp.int32),
                       pltpu.VMEM((per_w, H), jnp.float32),
                       pltpu.SemaphoreType.DMA,
                       pltpu.SemaphoreType.DMA],
    )
    def k(x_hbm, p0_hbm, p1_hbm, o_hbm, p0_v, p1_v, rows_v, s0, s1):
        wid = lax.axis_index("subcore") * NC + lax.axis_index("core")
        base = wid * per_w
        pltpu.sync_copy(p0_hbm.at[pl.ds(base, per_w)], p0_v)
        pltpu.sync_copy(p1_hbm.at[pl.ds(base, per_w)], p1_v)
        pltpu.sync_copy(x_hbm.at[pl.ds(base, per_w)], rows_v)
        pltpu.async_copy(rows_v, o_hbm.at[p0_v], s0)
        pltpu.async_copy(rows_v, o_hbm.at[p1_v], s1)
        pltpu.make_async_copy(rows_v, o_hbm.at[p0_v], s0).wait()
        pltpu.make_async_copy(rows_v, o_hbm.at[p1_v], s1).wait()

    return k(x, p0, p1)


def _sc_combine(ys, p0, p1, w0, w1):
    T_ = p0.shape[0]
    per_w = T_ // NW
    C = 8
    nch = per_w // C
    L = 16
    mesh = plsc.VectorSubcoreMesh(core_axis_name="core", subcore_axis_name="subcore")

    @functools.partial(
        pl.kernel,
        out_type=jax.ShapeDtypeStruct((T_, H), ys.dtype),
        mesh=mesh,
        scratch_types=[pltpu.VMEM((nch, C), jnp.int32),
                       pltpu.VMEM((nch, C), jnp.int32),
                       pltpu.VMEM((per_w, 16), jnp.float32),
                       pltpu.VMEM((per_w, 16), jnp.float32),
                       pltpu.VMEM((3, C, H), jnp.float32),
                       pltpu.VMEM((3, C, H), jnp.float32),
                       pltpu.VMEM((C, H), jnp.float32),
                       pltpu.SemaphoreType.DMA,
                       pltpu.SemaphoreType.DMA,
                       pltpu.SemaphoreType.DMA,
                       pltpu.SemaphoreType.DMA,
                       pltpu.SemaphoreType.DMA,
                       pltpu.SemaphoreType.DMA],
    )
    def k(ys_hbm, p0_hbm, p1_hbm, w0_hbm, w1_hbm, o_hbm,
          p0_v, p1_v, w0_v, w1_v, a_v, b_v, o_v, s0, s1, s2, s3, s4, s5):
        wid = lax.axis_index("subcore") * NC + lax.axis_index("core")
        base = wid * per_w
        pltpu.sync_copy(p0_hbm.at[wid], p0_v)
        pltpu.sync_copy(p1_hbm.at[wid], p1_v)
        pltpu.sync_copy(w0_hbm.at[pl.ds(base, per_w)], w0_v)
        pltpu.sync_copy(w1_hbm.at[pl.ds(base, per_w)], w1_v)
        sems = [(s0, s1), (s2, s3), (s4, s5)]
        NB = 3

        def fire(c):
            sa, sb = sems[c % NB]
            pltpu.async_copy(ys_hbm.at[p0_v.at[c]], a_v.at[c % NB], sa)
            pltpu.async_copy(ys_hbm.at[p1_v.at[c]], b_v.at[c % NB], sb)

        for c in range(min(NB, nch)):
            fire(c)
        for c in range(nch):
            sa, sb = sems[c % NB]
            pltpu.make_async_copy(ys_hbm.at[p0_v.at[c]], a_v.at[c % NB], sa).wait()
            pltpu.make_async_copy(ys_hbm.at[p1_v.at[c]], b_v.at[c % NB], sb).wait()

            @pl.loop(0, C)
            def _(r):
                wa = w0_v[c * C + r, :]
                wb = w1_v[c * C + r, :]

                @pl.loop(0, H, step=L)
                def _(col):
                    o_v[r, pl.ds(col, L)] = (
                        a_v[c % NB, r, pl.ds(col, L)] * wa
                        + b_v[c % NB, r, pl.ds(col, L)] * wb)

            pltpu.sync_copy(o_v, o_hbm.at[pl.ds(base + c * C, C)])
            if c + NB < nch:
                fire(c + NB)

    return k(ys, p0.reshape(NW, nch, C), p1.reshape(NW, nch, C), w0, w1)


def _tc_gmm_body(te_ref, nt_ref, xs_ref, g_ref, u_ref, d_ref, ys_ref):
    i = pl.program_id(0)

    @pl.when(i < nt_ref[0])
    def _():
        x = xs_ref[...]
        gate = jnp.dot(x, g_ref[...], preferred_element_type=jnp.float32)
        up = jnp.dot(x, u_ref[...], preferred_element_type=jnp.float32)
        h = (gate * jax.nn.sigmoid(gate)) * up
        ys_ref[...] = jnp.dot(h, d_ref[...], preferred_element_type=jnp.float32)


def _tc_gmm(xs, gate_up_proj, down_proj, te, nt, tiles, interpret=False):
    grid_spec = pltpu.PrefetchScalarGridSpec(
        num_scalar_prefetch=2,
        grid=(tiles,),
        in_specs=[
            pl.BlockSpec((M, H),
                         lambda i, te, nt: (jnp.minimum(i, nt[0] - 1), 0)),
            pl.BlockSpec((H, I), lambda i, te, nt: (te[i], 0)),
            pl.BlockSpec((H, I), lambda i, te, nt: (te[i], 1)),
            pl.BlockSpec((I, H), lambda i, te, nt: (te[i], 0)),
        ],
        out_specs=pl.BlockSpec(
            (M, H), lambda i, te, nt: (jnp.minimum(i, nt[0] - 1), 0)),
    )
    return pl.pallas_call(
        _tc_gmm_body,
        grid_spec=grid_spec,
        out_shape=jax.ShapeDtypeStruct((tiles * M, H), jnp.float32),
        compiler_params=pltpu.CompilerParams(
            dimension_semantics=("arbitrary",)),
        interpret=interpret,
    )(te, nt, xs, gate_up_proj, gate_up_proj, down_proj)


def kernel(hidden_states, topk_weights, topk_indices, gate_up_proj, down_proj):
    T_, K_ = topk_indices.shape
    N = T_ * K_
    tiles = N // M + E
    invpos, te, nt = _route(topk_indices, tiles)
    pos = invpos.reshape(T_, K_)
    p0 = pos[:, 0] + 0
    p1 = pos[:, 1] + 0
    w0 = jnp.broadcast_to(topk_weights[:, 0:1], (T_, 16)) + 0.0
    w1 = jnp.broadcast_to(topk_weights[:, 1:2], (T_, 16)) + 0.0
    xs = _sc_disperse(hidden_states, p0, p1, tiles * M)
    ys = _tc_gmm(xs, gate_up_proj, down_proj, te, nt, tiles)
    out = _sc_combine(ys, p0, p1, w0, w1)
    return out.astype(hidden_states.dtype)


# speedup vs baseline: 2.9354x; 1.0749x over previous
import functools

import jax
import jax.numpy as jnp
from jax import lax
from jax.experimental import pallas as pl
from jax.experimental.pallas import tpu as pltpu
from jax.experimental.pallas import tpu_sc as plsc

NC = 2
NS = 16
NW = NC * NS

E = 16
H = 1024
I = 1024
M = 512


def _route(topk_indices, tiles):
    T_, K_ = topk_indices.shape
    N = T_ * K_
    e_flat = topk_indices.reshape(N).astype(jnp.int32)
    oneh = (e_flat[:, None] == jnp.arange(E, dtype=jnp.int32)[None, :])
    counts = jnp.cumsum(oneh.astype(jnp.int16), axis=0).astype(jnp.int32)
    g = counts[-1]
    padded = ((g + (M - 1)) // M) * M
    ends = jnp.cumsum(padded).astype(jnp.int32)
    base = ends - padded
    rank = jnp.sum(jnp.where(oneh, counts, 0), axis=1) - 1
    slot_base = jnp.sum(jnp.where(oneh, base[None, :], 0), axis=1)
    invpos = (slot_base + rank).astype(jnp.int32)
    tile_starts = jnp.arange(tiles, dtype=jnp.int32) * M
    te = jnp.sum((ends[None, :] <= tile_starts[:, None]).astype(jnp.int32),
                 axis=1)
    last_e = jnp.sum((ends <= ends[-1] - 1).astype(jnp.int32))
    te = jnp.where(tile_starts < ends[-1], jnp.clip(te, 0, E - 1), last_e)
    nt = (ends[-1:] // M).astype(jnp.int32)
    return invpos, te.astype(jnp.int32), nt


def _sc_disperse(x, p0, p1, pad_n):
    T_ = p0.shape[0]
    per_w = T_ // NW
    mesh = plsc.VectorSubcoreMesh(core_axis_name="core", subcore_axis_name="subcore")

    @functools.partial(
        pl.kernel,
        out_type=jax.ShapeDtypeStruct((pad_n, H), x.dtype),
        mesh=mesh,
        scratch_types=[pltpu.VMEM((per_w,), jnp.int32),
                       pltpu.VMEM((per_w,), jnp.int32),
                       pltpu.VMEM((per_w, H), jnp.float32),
                       pltpu.SemaphoreType.DMA,
                       pltpu.SemaphoreType.DMA],
    )
    def k(x_hbm, p0_hbm, p1_hbm, o_hbm, p0_v, p1_v, rows_v, s0, s1):
        wid = lax.axis_index("subcore") * NC + lax.axis_index("core")
        base = wid * per_w
        pltpu.sync_copy(p0_hbm.at[pl.ds(base, per_w)], p0_v)
        pltpu.sync_copy(p1_hbm.at[pl.ds(base, per_w)], p1_v)
        pltpu.sync_copy(x_hbm.at[pl.ds(base, per_w)], rows_v)
        pltpu.async_copy(rows_v, o_hbm.at[p0_v], s0)
        pltpu.async_copy(rows_v, o_hbm.at[p1_v], s1)
        pltpu.make_async_copy(rows_v, o_hbm.at[p0_v], s0).wait()
        pltpu.make_async_copy(rows_v, o_hbm.at[p1_v], s1).wait()

    return k(x, p0, p1)


def _sc_combine(ys, p0, p1, w0, w1):
    T_ = p0.shape[0]
    per_w = T_ // NW
    C = 8
    nch = per_w // C
    L = 16
    mesh = plsc.VectorSubcoreMesh(core_axis_name="core", subcore_axis_name="subcore")

    @functools.partial(
        pl.kernel,
        out_type=jax.ShapeDtypeStruct((T_, H), ys.dtype),
        mesh=mesh,
        scratch_types=[pltpu.VMEM((nch, C), jnp.int32),
                       pltpu.VMEM((nch, C), jnp.int32),
                       pltpu.VMEM((per_w, 16), jnp.float32),
                       pltpu.VMEM((per_w, 16), jnp.float32),
                       pltpu.VMEM((3, C, H), jnp.float32),
                       pltpu.VMEM((3, C, H), jnp.float32),
                       pltpu.VMEM((C, H), jnp.float32),
                       pltpu.SemaphoreType.DMA,
                       pltpu.SemaphoreType.DMA,
                       pltpu.SemaphoreType.DMA,
                       pltpu.SemaphoreType.DMA,
                       pltpu.SemaphoreType.DMA,
                       pltpu.SemaphoreType.DMA],
    )
    def k(ys_hbm, p0_hbm, p1_hbm, w0_hbm, w1_hbm, o_hbm,
          p0_v, p1_v, w0_v, w1_v, a_v, b_v, o_v, s0, s1, s2, s3, s4, s5):
        wid = lax.axis_index("subcore") * NC + lax.axis_index("core")
        base = wid * per_w
        pltpu.sync_copy(p0_hbm.at[wid], p0_v)
        pltpu.sync_copy(p1_hbm.at[wid], p1_v)
        pltpu.sync_copy(w0_hbm.at[pl.ds(base, per_w)], w0_v)
        pltpu.sync_copy(w1_hbm.at[pl.ds(base, per_w)], w1_v)
        sems = [(s0, s1), (s2, s3), (s4, s5)]
        NB = 3

        def fire(c):
            sa, sb = sems[c % NB]
            pltpu.async_copy(ys_hbm.at[p0_v.at[c]], a_v.at[c % NB], sa)
            pltpu.async_copy(ys_hbm.at[p1_v.at[c]], b_v.at[c % NB], sb)

        for c in range(min(NB, nch)):
            fire(c)
        for c in range(nch):
            sa, sb = sems[c % NB]
            pltpu.make_async_copy(ys_hbm.at[p0_v.at[c]], a_v.at[c % NB], sa).wait()
            pltpu.make_async_copy(ys_hbm.at[p1_v.at[c]], b_v.at[c % NB], sb).wait()

            @pl.loop(0, C)
            def _(r):
                wa = w0_v[c * C + r, :]
                wb = w1_v[c * C + r, :]
                for col in range(0, H, L):
                    o_v[r, pl.ds(col, L)] = (
                        a_v[c % NB, r, pl.ds(col, L)] * wa
                        + b_v[c % NB, r, pl.ds(col, L)] * wb)

            pltpu.sync_copy(o_v, o_hbm.at[pl.ds(base + c * C, C)])
            if c + NB < nch:
                fire(c + NB)

    return k(ys, p0.reshape(NW, nch, C), p1.reshape(NW, nch, C), w0, w1)


def _tc_gmm_body(te_ref, nt_ref, xs_ref, g_ref, u_ref, d_ref, ys_ref):
    i = pl.program_id(0)

    @pl.when(i < nt_ref[0])
    def _():
        x = xs_ref[...]
        gate = jnp.dot(x, g_ref[...], preferred_element_type=jnp.float32)
        up = jnp.dot(x, u_ref[...], preferred_element_type=jnp.float32)
        h = (gate * jax.nn.sigmoid(gate)) * up
        ys_ref[...] = jnp.dot(h, d_ref[...], preferred_element_type=jnp.float32)


def _tc_gmm(xs, gate_up_proj, down_proj, te, nt, tiles, interpret=False):
    grid_spec = pltpu.PrefetchScalarGridSpec(
        num_scalar_prefetch=2,
        grid=(tiles,),
        in_specs=[
            pl.BlockSpec((M, H),
                         lambda i, te, nt: (jnp.minimum(i, nt[0] - 1), 0)),
            pl.BlockSpec((H, I), lambda i, te, nt: (te[i], 0)),
            pl.BlockSpec((H, I), lambda i, te, nt: (te[i], 1)),
            pl.BlockSpec((I, H), lambda i, te, nt: (te[i], 0)),
        ],
        out_specs=pl.BlockSpec(
            (M, H), lambda i, te, nt: (jnp.minimum(i, nt[0] - 1), 0)),
    )
    return pl.pallas_call(
        _tc_gmm_body,
        grid_spec=grid_spec,
        out_shape=jax.ShapeDtypeStruct((tiles * M, H), jnp.float32),
        compiler_params=pltpu.CompilerParams(
            dimension_semantics=("arbitrary",)),
        interpret=interpret,
    )(te, nt, xs, gate_up_proj, gate_up_proj, down_proj)


def kernel(hidden_states, topk_weights, topk_indices, gate_up_proj, down_proj):
    T_, K_ = topk_indices.shape
    N = T_ * K_
    tiles = N // M + E
    invpos, te, nt = _route(topk_indices, tiles)
    pos = invpos.reshape(T_, K_)
    p0 = pos[:, 0] + 0
    p1 = pos[:, 1] + 0
    w0 = jnp.broadcast_to(topk_weights[:, 0:1], (T_, 16)) + 0.0
    w1 = jnp.broadcast_to(topk_weights[:, 1:2], (T_, 16)) + 0.0
    xs = _sc_disperse(hidden_states, p0, p1, tiles * M)
    ys = _tc_gmm(xs, gate_up_proj, down_proj, te, nt, tiles)
    out = _sc_combine(ys, p0, p1, w0, w1)
    return out.astype(hidden_states.dtype)


# speedup vs baseline: 2.9438x; 1.0028x over previous
import functools

import jax
import jax.numpy as jnp
from jax import lax
from jax.experimental import pallas as pl
from jax.experimental.pallas import tpu as pltpu
from jax.experimental.pallas import tpu_sc as plsc

NC = 2
NS = 16
NW = NC * NS

E = 16
H = 1024
I = 1024
M = 512


def _route(topk_indices, tiles):
    T_, K_ = topk_indices.shape
    N = T_ * K_
    e_flat = topk_indices.reshape(N).astype(jnp.int32)
    oneh = (e_flat[:, None] == jnp.arange(E, dtype=jnp.int32)[None, :])
    counts = jnp.cumsum(oneh.astype(jnp.int16), axis=0).astype(jnp.int32)
    g = counts[-1]
    padded = ((g + (M - 1)) // M) * M
    ends = jnp.cumsum(padded).astype(jnp.int32)
    base = ends - padded
    rank = jnp.sum(jnp.where(oneh, counts, 0), axis=1) - 1
    slot_base = jnp.sum(jnp.where(oneh, base[None, :], 0), axis=1)
    invpos = (slot_base + rank).astype(jnp.int32)
    tile_starts = jnp.arange(tiles, dtype=jnp.int32) * M
    te = jnp.sum((ends[None, :] <= tile_starts[:, None]).astype(jnp.int32),
                 axis=1)
    last_e = jnp.sum((ends <= ends[-1] - 1).astype(jnp.int32))
    te = jnp.where(tile_starts < ends[-1], jnp.clip(te, 0, E - 1), last_e)
    nt = (ends[-1:] // M).astype(jnp.int32)
    return invpos, te.astype(jnp.int32), nt


def _sc_disperse(x, p0, p1, pad_n):
    T_ = p0.shape[0]
    per_w = T_ // NW
    mesh = plsc.VectorSubcoreMesh(core_axis_name="core", subcore_axis_name="subcore")

    @functools.partial(
        pl.kernel,
        out_type=jax.ShapeDtypeStruct((pad_n, H), x.dtype),
        mesh=mesh,
        scratch_types=[pltpu.VMEM((per_w,), jnp.int32),
                       pltpu.VMEM((per_w,), jnp.int32),
                       pltpu.VMEM((per_w, H), jnp.float32),
                       pltpu.SemaphoreType.DMA,
                       pltpu.SemaphoreType.DMA],
    )
    def k(x_hbm, p0_hbm, p1_hbm, o_hbm, p0_v, p1_v, rows_v, s0, s1):
        wid = lax.axis_index("subcore") * NC + lax.axis_index("core")
        base = wid * per_w
        pltpu.sync_copy(p0_hbm.at[pl.ds(base, per_w)], p0_v)
        pltpu.sync_copy(p1_hbm.at[pl.ds(base, per_w)], p1_v)
        pltpu.sync_copy(x_hbm.at[pl.ds(base, per_w)], rows_v)
        pltpu.async_copy(rows_v, o_hbm.at[p0_v], s0)
        pltpu.async_copy(rows_v, o_hbm.at[p1_v], s1)
        pltpu.make_async_copy(rows_v, o_hbm.at[p0_v], s0).wait()
        pltpu.make_async_copy(rows_v, o_hbm.at[p1_v], s1).wait()

    return k(x, p0, p1)


def _sc_combine(ys, p0, p1, w0, w1):
    T_ = p0.shape[0]
    per_w = T_ // NW
    C = 8
    nch = per_w // C
    L = 16
    mesh = plsc.VectorSubcoreMesh(core_axis_name="core", subcore_axis_name="subcore")

    @functools.partial(
        pl.kernel,
        out_type=jax.ShapeDtypeStruct((T_, H), ys.dtype),
        mesh=mesh,
        scratch_types=[pltpu.VMEM((nch, C), jnp.int32),
                       pltpu.VMEM((nch, C), jnp.int32),
                       pltpu.VMEM((per_w, 16), jnp.float32),
                       pltpu.VMEM((per_w, 16), jnp.float32),
                       pltpu.VMEM((3, C, H), jnp.float32),
                       pltpu.VMEM((3, C, H), jnp.float32),
                       pltpu.VMEM((2, C, H), jnp.float32),
                       pltpu.SemaphoreType.DMA,
                       pltpu.SemaphoreType.DMA,
                       pltpu.SemaphoreType.DMA,
                       pltpu.SemaphoreType.DMA,
                       pltpu.SemaphoreType.DMA,
                       pltpu.SemaphoreType.DMA,
                       pltpu.SemaphoreType.DMA,
                       pltpu.SemaphoreType.DMA],
    )
    def k(ys_hbm, p0_hbm, p1_hbm, w0_hbm, w1_hbm, o_hbm,
          p0_v, p1_v, w0_v, w1_v, a_v, b_v, o_v, s0, s1, s2, s3, s4, s5, t0, t1):
        wid = lax.axis_index("subcore") * NC + lax.axis_index("core")
        base = wid * per_w
        pltpu.sync_copy(p0_hbm.at[wid], p0_v)
        pltpu.sync_copy(p1_hbm.at[wid], p1_v)
        pltpu.sync_copy(w0_hbm.at[pl.ds(base, per_w)], w0_v)
        pltpu.sync_copy(w1_hbm.at[pl.ds(base, per_w)], w1_v)
        sems = [(s0, s1), (s2, s3), (s4, s5)]
        tsems = [t0, t1]
        NB = 3

        def fire(c):
            sa, sb = sems[c % NB]
            pltpu.async_copy(ys_hbm.at[p0_v.at[c]], a_v.at[c % NB], sa)
            pltpu.async_copy(ys_hbm.at[p1_v.at[c]], b_v.at[c % NB], sb)

        def store_wait(c):
            pltpu.make_async_copy(
                o_v.at[c % 2], o_hbm.at[pl.ds(base + c * C, C)],
                tsems[c % 2]).wait()

        for c in range(min(NB, nch)):
            fire(c)
        for c in range(nch):
            sa, sb = sems[c % NB]
            pltpu.make_async_copy(ys_hbm.at[p0_v.at[c]], a_v.at[c % NB], sa).wait()
            pltpu.make_async_copy(ys_hbm.at[p1_v.at[c]], b_v.at[c % NB], sb).wait()
            if c >= 2:
                store_wait(c - 2)

            @pl.loop(0, C)
            def _(r):
                wa = w0_v[c * C + r, :]
                wb = w1_v[c * C + r, :]
                for col in range(0, H, L):
                    o_v[c % 2, r, pl.ds(col, L)] = (
                        a_v[c % NB, r, pl.ds(col, L)] * wa
                        + b_v[c % NB, r, pl.ds(col, L)] * wb)

            pltpu.async_copy(o_v.at[c % 2], o_hbm.at[pl.ds(base + c * C, C)],
                             tsems[c % 2])
            if c + NB < nch:
                fire(c + NB)
        for c in range(max(nch - 2, 0), nch):
            store_wait(c)

    return k(ys, p0.reshape(NW, nch, C), p1.reshape(NW, nch, C), w0, w1)


def _tc_gmm_body(te_ref, nt_ref, xs_ref, g_ref, u_ref, d_ref, ys_ref):
    i = pl.program_id(0)

    @pl.when(i < nt_ref[0])
    def _():
        x = xs_ref[...]
        gate = jnp.dot(x, g_ref[...], preferred_element_type=jnp.float32)
        up = jnp.dot(x, u_ref[...], preferred_element_type=jnp.float32)
        h = (gate * jax.nn.sigmoid(gate)) * up
        ys_ref[...] = jnp.dot(h, d_ref[...], preferred_element_type=jnp.float32)


def _tc_gmm(xs, gate_up_proj, down_proj, te, nt, tiles, interpret=False):
    grid_spec = pltpu.PrefetchScalarGridSpec(
        num_scalar_prefetch=2,
        grid=(tiles,),
        in_specs=[
            pl.BlockSpec((M, H),
                         lambda i, te, nt: (jnp.minimum(i, nt[0] - 1), 0)),
            pl.BlockSpec((H, I), lambda i, te, nt: (te[i], 0)),
            pl.BlockSpec((H, I), lambda i, te, nt: (te[i], 1)),
            pl.BlockSpec((I, H), lambda i, te, nt: (te[i], 0)),
        ],
        out_specs=pl.BlockSpec(
            (M, H), lambda i, te, nt: (jnp.minimum(i, nt[0] - 1), 0)),
    )
    return pl.pallas_call(
        _tc_gmm_body,
        grid_spec=grid_spec,
        out_shape=jax.ShapeDtypeStruct((tiles * M, H), jnp.float32),
        compiler_params=pltpu.CompilerParams(
            dimension_semantics=("arbitrary",)),
        interpret=interpret,
    )(te, nt, xs, gate_up_proj, gate_up_proj, down_proj)


def kernel(hidden_states, topk_weights, topk_indices, gate_up_proj, down_proj):
    T_, K_ = topk_indices.shape
    N = T_ * K_
    tiles = N // M + E
    invpos, te, nt = _route(topk_indices, tiles)
    pos = invpos.reshape(T_, K_)
    p0 = pos[:, 0] + 0
    p1 = pos[:, 1] + 0
    w0 = jnp.broadcast_to(topk_weights[:, 0:1], (T_, 16)) + 0.0
    w1 = jnp.broadcast_to(topk_weights[:, 1:2], (T_, 16)) + 0.0
    xs = _sc_disperse(hidden_states, p0, p1, tiles * M)
    ys = _tc_gmm(xs, gate_up_proj, down_proj, te, nt, tiles)
    out = _sc_combine(ys, p0, p1, w0, w1)
    return out.astype(hidden_states.dtype)
